# Initial kernel scaffold; baseline (speedup 1.0000x reference)
import functools

import jax
import jax.numpy as jnp
from jax import lax
from jax.experimental import pallas as pl
from jax.experimental.pallas import tpu as pltpu
from jax.experimental.pallas import tpu_sc as plsc

BATCH = 16384
NUM_KCS = 1000
NUM_CORES = 2
NUM_SUBCORES = 16
NW = NUM_CORES * NUM_SUBCORES
BPW = BATCH // NW
L = 16
CHUNKS = BPW // L
EPSILON = 1e-8


def _sigmoid(x):
    return 1.0 / (1.0 + jnp.exp(-x))


def _body(h_hbm, obs_hbm, pT_hbm, pF_hbm, pG_hbm, pS_hbm,
          omega_hbm, sigma_hbm, theta_hbm, kc_hbm, pid_hbm, sid_hbm,
          hnew_hbm, pcorr_hbm,
          pT_v, pF_v, pG_v, pS_v,
          kc_v, pid_v, sid_v,
          om_v, sg_v, th_v, h_v, obs_v, hn_v, pc_v, sem):
    c = lax.axis_index("c")
    s = lax.axis_index("s")
    wid = s * NUM_CORES + c
    base = wid * BPW

    pltpu.sync_copy(kc_hbm.at[pl.ds(base, BPW)], kc_v)
    pltpu.sync_copy(pid_hbm.at[pl.ds(base, BPW)], pid_v)
    pltpu.sync_copy(sid_hbm.at[pl.ds(base, BPW)], sid_v)
    pltpu.sync_copy(h_hbm.at[pl.ds(base, BPW)], h_v)
    pltpu.sync_copy(obs_hbm.at[pl.ds(base, BPW)], obs_v)

    pltpu.sync_copy(pT_hbm, pT_v)
    pltpu.sync_copy(pF_hbm, pF_v)
    pltpu.sync_copy(pG_hbm, pG_v)
    pltpu.sync_copy(pS_hbm, pS_v)

    cp_om = pltpu.async_copy(omega_hbm.at[pid_v], om_v, sem)
    cp_sg = pltpu.async_copy(sigma_hbm.at[pid_v], sg_v, sem)
    cp_th = pltpu.async_copy(theta_hbm.at[sid_v], th_v, sem)
    cp_om.wait()
    cp_sg.wait()
    cp_th.wait()

    zeros = jnp.zeros((L,), jnp.int32)
    ones = jnp.ones((L,), jnp.int32)
    twos = jnp.full((L,), 2, jnp.int32)
    threes = jnp.full((L,), 3, jnp.int32)

    def step(i, carry):
        off = i * L
        rows = off + lax.iota(jnp.int32, L)
        kc = kc_v[pl.ds(off, L)]
        pT_l = plsc.load_gather(pT_v, [kc])
        pF_l = plsc.load_gather(pF_v, [kc])
        pG_l = plsc.load_gather(pG_v, [kc])
        pS_l = plsc.load_gather(pS_v, [kc])
        om = om_v[pl.ds(off, L)]
        sg = sg_v[pl.ds(off, L)]
        th_L = plsc.load_gather(th_v, [rows, zeros])
        th_nF = plsc.load_gather(th_v, [rows, ones])
        th_G = plsc.load_gather(th_v, [rows, twos])
        th_nS = plsc.load_gather(th_v, [rows, threes])
        h0 = plsc.load_gather(h_v, [rows, zeros])
        h1 = plsc.load_gather(h_v, [rows, ones])
        obs = obs_v[pl.ds(off, L)]

        pT = _sigmoid(pT_l + th_L)
        pF = _sigmoid(pF_l - th_nF)
        pG = _sigmoid(pG_l + om + th_G)
        pS = _sigmoid(pS_l + sg - th_nS)
        obs_b = obs > 0.5
        p_m = jnp.where(obs_b, 1.0 - pS, pS)
        p_u = jnp.where(obs_b, pG, 1.0 - pG)
        a_u = p_u * h0
        a_m = p_m * h1
        nm = (1.0 - pF) * a_m + pT * a_u
        nu = pF * a_m + (1.0 - pT) * a_u
        inv = 1.0 / (nm + nu + EPSILON)
        nm = nm * inv
        nu = nu * inv
        pc = (1.0 - pS) * nm + pG * nu

        plsc.store_scatter(hn_v, [rows, zeros], nu)
        plsc.store_scatter(hn_v, [rows, ones], nm)
        pc_v[pl.ds(off, L)] = pc
        return carry

    lax.fori_loop(0, CHUNKS, step, 0)

    pltpu.sync_copy(hn_v, hnew_hbm.at[pl.ds(base, BPW)])
    pltpu.sync_copy(pc_v, pcorr_hbm.at[pl.ds(base, BPW)])


@jax.jit
def _run(h_prev, observation, pT_logit, pF_logit, pG_logit, pS_logit,
         omega, sigma, theta, kc_ids, pid, sid):
    mesh = plsc.VectorSubcoreMesh(
        core_axis_name="c", subcore_axis_name="s",
        num_cores=NUM_CORES, num_subcores=NUM_SUBCORES)
    f = pl.kernel(
        _body,
        out_type=(
            jax.ShapeDtypeStruct((BATCH, 2), jnp.float32),
            jax.ShapeDtypeStruct((BATCH,), jnp.float32),
        ),
        mesh=mesh,
        scratch_types=[
            pltpu.VMEM((NUM_KCS,), jnp.float32),
            pltpu.VMEM((NUM_KCS,), jnp.float32),
            pltpu.VMEM((NUM_KCS,), jnp.float32),
            pltpu.VMEM((NUM_KCS,), jnp.float32),
            pltpu.VMEM((BPW,), jnp.int32),
            pltpu.VMEM((BPW,), jnp.int32),
            pltpu.VMEM((BPW,), jnp.int32),
            pltpu.VMEM((BPW,), jnp.float32),
            pltpu.VMEM((BPW,), jnp.float32),
            pltpu.VMEM((BPW, 4), jnp.float32),
            pltpu.VMEM((BPW, 2), jnp.float32),
            pltpu.VMEM((BPW,), jnp.float32),
            pltpu.VMEM((BPW, 2), jnp.float32),
            pltpu.VMEM((BPW,), jnp.float32),
            pltpu.SemaphoreType.DMA,
        ],
        name="bkt_irt_sc",
    )
    return f(h_prev, observation, pT_logit, pF_logit, pG_logit, pS_logit,
             omega, sigma, theta, kc_ids, pid, sid)


def kernel(h_prev, observation, pT_logit, pF_logit, pG_logit, pS_logit,
           omega_w, sigma_w, student_ability_w, kc_ids, problem_ids,
           student_ids):
    kc = kc_ids.astype(jnp.int32)
    pid = problem_ids.astype(jnp.int32)
    sid = student_ids.astype(jnp.int32)
    omega = omega_w.reshape(-1)
    sigma = sigma_w.reshape(-1)
    h_new, p_correct = _run(
        h_prev, observation, pT_logit, pF_logit, pG_logit, pS_logit,
        omega, sigma, student_ability_w, kc, pid, sid)
    return (h_new, p_correct)


# speedup vs baseline: 3.6317x; 3.6317x over previous
import functools

import jax
import jax.numpy as jnp
from jax import lax
from jax.experimental import pallas as pl
from jax.experimental.pallas import tpu as pltpu
from jax.experimental.pallas import tpu_sc as plsc

BATCH = 16384
NUM_KCS = 1000
NUM_CORES = 2
NUM_SUBCORES = 16
NW = NUM_CORES * NUM_SUBCORES
BPW = BATCH // NW
L = 16
CHUNKS = BPW // L
EPSILON = 1e-8


def _sigmoid(x):
    return 1.0 / (1.0 + jnp.exp(-x))


def _body(h_hbm, obs_hbm, pT_hbm, pF_hbm, pG_hbm, pS_hbm,
          omega_hbm, sigma_hbm, theta_hbm, kc_hbm, pid_hbm, sid_hbm,
          hnew_hbm, pcorr_hbm,
          pT_v, pF_v, pG_v, pS_v,
          kc_v, pid_v, sid_v,
          ti0_v, ti1_v, ti2_v, ti3_v,
          om_v, sg_v, th0_v, th1_v, th2_v, th3_v,
          h_v, obs_v, hn_v, pc_v, sem):
    c = lax.axis_index("c")
    s = lax.axis_index("s")
    wid = s * NUM_CORES + c
    base = wid * BPW

    pltpu.sync_copy(kc_hbm.at[pl.ds(base, BPW)], kc_v)
    pltpu.sync_copy(pid_hbm.at[pl.ds(base, BPW)], pid_v)
    pltpu.sync_copy(sid_hbm.at[pl.ds(base, BPW)], sid_v)
    pltpu.sync_copy(h_hbm.at[pl.ds(2 * base, 2 * BPW)], h_v)
    pltpu.sync_copy(obs_hbm.at[pl.ds(base, BPW)], obs_v)

    pltpu.sync_copy(pT_hbm, pT_v)
    pltpu.sync_copy(pF_hbm, pF_v)
    pltpu.sync_copy(pG_hbm, pG_v)
    pltpu.sync_copy(pS_hbm, pS_v)

    def mkidx(i, carry):
        off = i * L
        s4 = sid_v[pl.ds(off, L)] * 4
        ti0_v[pl.ds(off, L)] = s4
        ti1_v[pl.ds(off, L)] = s4 + 1
        ti2_v[pl.ds(off, L)] = s4 + 2
        ti3_v[pl.ds(off, L)] = s4 + 3
        return carry

    lax.fori_loop(0, CHUNKS, mkidx, 0)

    cp_om = pltpu.async_copy(omega_hbm.at[pid_v], om_v, sem)
    cp_sg = pltpu.async_copy(sigma_hbm.at[pid_v], sg_v, sem)
    cp_t0 = pltpu.async_copy(theta_hbm.at[ti0_v], th0_v, sem)
    cp_t1 = pltpu.async_copy(theta_hbm.at[ti1_v], th1_v, sem)
    cp_t2 = pltpu.async_copy(theta_hbm.at[ti2_v], th2_v, sem)
    cp_t3 = pltpu.async_copy(theta_hbm.at[ti3_v], th3_v, sem)
    cp_om.wait()
    cp_sg.wait()
    cp_t0.wait()
    cp_t1.wait()
    cp_t2.wait()
    cp_t3.wait()

    def step(i, carry):
        off = i * L
        rows = off + lax.iota(jnp.int32, L)
        rows2 = rows * 2
        kc = kc_v[pl.ds(off, L)]
        pT_l = plsc.load_gather(pT_v, [kc])
        pF_l = plsc.load_gather(pF_v, [kc])
        pG_l = plsc.load_gather(pG_v, [kc])
        pS_l = plsc.load_gather(pS_v, [kc])
        om = om_v[pl.ds(off, L)]
        sg = sg_v[pl.ds(off, L)]
        th_L = th0_v[pl.ds(off, L)]
        th_nF = th1_v[pl.ds(off, L)]
        th_G = th2_v[pl.ds(off, L)]
        th_nS = th3_v[pl.ds(off, L)]
        h0 = plsc.load_gather(h_v, [rows2])
        h1 = plsc.load_gather(h_v, [rows2 + 1])
        obs = obs_v[pl.ds(off, L)]

        pT = _sigmoid(pT_l + th_L)
        pF = _sigmoid(pF_l - th_nF)
        pG = _sigmoid(pG_l + om + th_G)
        pS = _sigmoid(pS_l + sg - th_nS)
        obs_b = obs > 0.5
        p_m = jnp.where(obs_b, 1.0 - pS, pS)
        p_u = jnp.where(obs_b, pG, 1.0 - pG)
        a_u = p_u * h0
        a_m = p_m * h1
        nm = (1.0 - pF) * a_m + pT * a_u
        nu = pF * a_m + (1.0 - pT) * a_u
        inv = 1.0 / (nm + nu + EPSILON)
        nm = nm * inv
        nu = nu * inv
        pc = (1.0 - pS) * nm + pG * nu

        plsc.store_scatter(hn_v, [rows2], nu)
        plsc.store_scatter(hn_v, [rows2 + 1], nm)
        pc_v[pl.ds(off, L)] = pc
        return carry

    lax.fori_loop(0, CHUNKS, step, 0)

    pltpu.sync_copy(hn_v, hnew_hbm.at[pl.ds(2 * base, 2 * BPW)])
    pltpu.sync_copy(pc_v, pcorr_hbm.at[pl.ds(base, BPW)])


@jax.jit
def _run(h_prev, observation, pT_logit, pF_logit, pG_logit, pS_logit,
         omega, sigma, theta, kc_ids, pid, sid):
    mesh = plsc.VectorSubcoreMesh(
        core_axis_name="c", subcore_axis_name="s",
        num_cores=NUM_CORES, num_subcores=NUM_SUBCORES)
    f = pl.kernel(
        _body,
        out_type=(
            jax.ShapeDtypeStruct((2 * BATCH,), jnp.float32),
            jax.ShapeDtypeStruct((BATCH,), jnp.float32),
        ),
        mesh=mesh,
        scratch_types=[
            pltpu.VMEM((NUM_KCS,), jnp.float32),
            pltpu.VMEM((NUM_KCS,), jnp.float32),
            pltpu.VMEM((NUM_KCS,), jnp.float32),
            pltpu.VMEM((NUM_KCS,), jnp.float32),
            pltpu.VMEM((BPW,), jnp.int32),
            pltpu.VMEM((BPW,), jnp.int32),
            pltpu.VMEM((BPW,), jnp.int32),
            pltpu.VMEM((BPW,), jnp.int32),
            pltpu.VMEM((BPW,), jnp.int32),
            pltpu.VMEM((BPW,), jnp.int32),
            pltpu.VMEM((BPW,), jnp.int32),
            pltpu.VMEM((BPW,), jnp.float32),
            pltpu.VMEM((BPW,), jnp.float32),
            pltpu.VMEM((BPW,), jnp.float32),
            pltpu.VMEM((BPW,), jnp.float32),
            pltpu.VMEM((BPW,), jnp.float32),
            pltpu.VMEM((BPW,), jnp.float32),
            pltpu.VMEM((2 * BPW,), jnp.float32),
            pltpu.VMEM((BPW,), jnp.float32),
            pltpu.VMEM((2 * BPW,), jnp.float32),
            pltpu.VMEM((BPW,), jnp.float32),
            pltpu.SemaphoreType.DMA,
        ],
        compiler_params=pltpu.CompilerParams(needs_layout_passes=False),
        name="bkt_irt_sc",
    )
    return f(h_prev, observation, pT_logit, pF_logit, pG_logit, pS_logit,
             omega, sigma, theta, kc_ids, pid, sid)


def kernel(h_prev, observation, pT_logit, pF_logit, pG_logit, pS_logit,
           omega_w, sigma_w, student_ability_w, kc_ids, problem_ids,
           student_ids):
    kc = kc_ids.astype(jnp.int32)
    pid = problem_ids.astype(jnp.int32)
    sid = student_ids.astype(jnp.int32)
    omega = omega_w.reshape(-1)
    sigma = sigma_w.reshape(-1)
    theta = student_ability_w.reshape(-1)
    h_flat = h_prev.reshape(-1)
    h_new_flat, p_correct = _run(
        h_flat, observation, pT_logit, pF_logit, pG_logit, pS_logit,
        omega, sigma, theta, kc, pid, sid)
    return (h_new_flat.reshape(BATCH, 2), p_correct)
